# baseline (speedup 1.0000x reference)
import jax
import jax.numpy as jnp
import numpy as np
from jax.experimental import pallas as pl

B, C_IN, N, K_NN, CH, N_BLOCKS = 4, 3, 2048, 16, 64, 7


def _knn_idx(x, k, d):
    xt = jnp.transpose(jnp.squeeze(x, -1), (0, 2, 1))
    sq = jnp.sum(xt * xt, axis=-1)
    dist = sq[:, :, None] - 2.0 * jnp.einsum('bnc,bmc->bnm', xt, xt) + sq[:, None, :]
    _, idx = jax.lax.top_k(-dist, k * d)
    return idx[:, :, ::d]


def _batch_norm(x, g, b):
    mean = jnp.mean(x, axis=(0, 2, 3), keepdims=True)
    var = jnp.var(x, axis=(0, 2, 3), keepdims=True)
    xn = (x - mean) / jnp.sqrt(var + 1e-5)
    return xn * g[None, :, None, None] + b[None, :, None, None]


def _edge_conv(x, idx, p):
    W, b, g, be = p
    xb = jnp.transpose(jnp.squeeze(x, -1), (0, 2, 1))
    xj = jax.vmap(lambda f, i: f[i])(xb, idx)
    xi = jnp.broadcast_to(xb[:, :, None, :], xj.shape)
    feat = jnp.concatenate([xi, xj - xi], axis=-1)
    out = jnp.einsum('bnkc,oc->bonk', feat, W) + b[None, :, None, None]
    out = jax.nn.relu(_batch_norm(out, g, be))
    out = jnp.max(out, axis=-1, keepdims=True)
    return out


def kernel(inputs, params):
    idx0 = _knn_idx(inputs[:, 0:3], K_NN, 1)
    feats = [_edge_conv(inputs, idx0, params['head'])]
    x = feats[0]
    for i in range(N_BLOCKS - 1):
        d = i % 4 + 1
        idx = _knn_idx(x, K_NN, d)
        x = _edge_conv(x, idx, params['blocks'][i]) + x
        feats.append(x)
    feats = jnp.concatenate(feats, axis=1)
    Wf, bf, gf, bef = params['fusion']
    fus = jnp.einsum('oc,bcnk->bonk', Wf, feats) + bf[None, :, None, None]
    fus = jax.nn.relu(_batch_norm(fus, gf, bef))
    fus = jnp.max(fus, axis=2, keepdims=True)
    fus = jnp.repeat(fus, feats.shape[2], axis=2)
    return jnp.concatenate([fus, feats], axis=1)


# speedup vs baseline: 3.5414x; 3.5414x over previous
import functools

import jax
import jax.numpy as jnp
import numpy as np
from jax import lax
from jax.experimental import pallas as pl
from jax.experimental.pallas import tpu as pltpu
from jax.experimental.pallas import tpu_sc as plsc

B, N, K, CH = 4, 2048, 16, 64
NB = 7
TR = 128
TP = 512
GSTEPS = B * (N // TP)
CNT_E = float(B * N * K)
CNT_F = float(B * N)
BIGF = np.float32(3.0e38)
BIGI = np.int32(1 << 30)


CS = 256
NCH = N // CS
NV = N // 8


def _dist_body(x_ref, xt_ref, d_ref):
    Xrt = xt_ref[0]
    c = Xrt.shape[0]
    x2 = Xrt * Xrt
    if c > 8:
        x2 = jnp.sum(x2.reshape(c // 8, 8, TR), axis=0)
    sqr = jnp.sum(x2, axis=0)
    parts = []
    for ch in range(NCH):
        Xc = x_ref[0, pl.ds(ch * CS, CS), :]
        dot = lax.dot_general(Xc, Xrt, (((1,), (0,)), ((), ())),
                              preferred_element_type=jnp.float32)
        sq = jnp.sum(Xc * Xc, axis=1, keepdims=True)
        parts.append((sqr[None, :] - 2.0 * dot) + sq)
    d_ref[0] = jnp.concatenate(parts, axis=0)


def _dist(x, xT):
    c = x.shape[-1]
    return pl.pallas_call(
        _dist_body,
        grid=(B, N // TR),
        in_specs=[
            pl.BlockSpec((1, N, c), lambda b, t: (b, 0, 0)),
            pl.BlockSpec((1, c, TR), lambda b, t: (b, 0, t)),
        ],
        out_specs=pl.BlockSpec((1, N, TR), lambda b, t: (b, 0, t)),
        out_shape=jax.ShapeDtypeStruct((B, N, N), jnp.float32),
    )(x, xT)


def _topk_body(d_ref, idx_ref, st_m, st_c, isc, *, d):
    kd = 15 * d + 1
    b = pl.program_id(0)
    i = pl.program_id(2)

    @pl.when(i == 0)
    def _init():
        st_m[...] = jnp.full((8, TR), -BIGF, jnp.float32)
        st_c[...] = jnp.full((8, TR), -1, jnp.int32)

    mp = st_m[...]
    cp = st_c[...]
    iota_s = lax.broadcasted_iota(jnp.int32, (8, TR), 0)
    M8 = jnp.full((8, TR), BIGF, jnp.float32)
    for r in range(NV):
        a = d_ref[0, pl.ds(r * 8, 8), :]
        idv = iota_s + r * 8
        elig = (a > mp) | ((a == mp) & (idv > cp))
        M8 = jnp.minimum(M8, jnp.where(elig, a, BIGF))
    mb = jnp.broadcast_to(jnp.min(M8, axis=0)[None, :], (8, TR))
    C8 = jnp.full((8, TR), BIGI, jnp.int32)
    for r in range(NV):
        a = d_ref[0, pl.ds(r * 8, 8), :]
        idv = iota_s + r * 8
        elig = (a > mp) | ((a == mp) & (idv > cp))
        C8 = jnp.minimum(C8, jnp.where(elig & (a == mb), idv, BIGI))
    cb = jnp.broadcast_to(jnp.min(C8, axis=0)[None, :], (8, TR))
    st_m[...] = mb
    st_c[...] = cb
    isc[pl.ds(i, 1), :] = cb[0:1, :]

    @pl.when(i == kd - 1)
    def _emit():
        boff = b * N
        for j in range(K):
            idx_ref[0, j, :] = isc[j * d, :] + boff


def _topk(dist, d):
    kd = 15 * d + 1
    kdpad = ((kd + 7) // 8) * 8
    return pl.pallas_call(
        functools.partial(_topk_body, d=d),
        grid=(B, N // TR, kd),
        in_specs=[pl.BlockSpec((1, N, TR), lambda b, t, i: (b, 0, t))],
        out_specs=pl.BlockSpec((1, K, TR), lambda b, t, i: (b, 0, t)),
        out_shape=jax.ShapeDtypeStruct((B, K, N), jnp.int32),
        scratch_shapes=[pltpu.VMEM((8, TR), jnp.float32),
                        pltpu.VMEM((8, TR), jnp.int32),
                        pltpu.VMEM((kdpad, TR), jnp.int32)],
    )(dist)


def _knn(x, d):
    xT = jnp.transpose(x, (0, 2, 1))
    return _topk(_dist(x, xT), d)


_L_TOT = B * K * N
_NW = 32
_PERW = _L_TOT // _NW
_CHUNK = 128


def _sc_gather_body(idx_hbm, table_hbm, out_hbm, idx_v, rows_v, sem):
    wid = lax.axis_index("s") * 2 + lax.axis_index("c")
    base = wid * _PERW

    def chunk(c, _):
        off = base + c * _CHUNK
        pltpu.sync_copy(idx_hbm.at[pl.ds(off, _CHUNK)], idx_v)
        pltpu.async_copy(table_hbm.at[idx_v], rows_v, sem).wait()
        pltpu.sync_copy(rows_v, out_hbm.at[pl.ds(off, _CHUNK)])
        return 0

    lax.fori_loop(0, _PERW // _CHUNK, chunk, 0)


_sc_gather_cache = []


def _gather(idx_flat, table):
    if not _sc_gather_cache:
        _sc_gather_cache.append(pl.kernel(
            _sc_gather_body,
            mesh=plsc.VectorSubcoreMesh(core_axis_name="c",
                                        subcore_axis_name="s"),
            out_type=jax.ShapeDtypeStruct((_L_TOT, 2 * CH), jnp.float32),
            scratch_types=[pltpu.VMEM((_CHUNK,), jnp.int32),
                           pltpu.VMEM((_CHUNK, 2 * CH), jnp.float32),
                           pltpu.SemaphoreType.DMA],
        ))
    return _sc_gather_cache[0](idx_flat, table)


def _c1_body(rows_ref, x_ref, wt_ref, bv_ref, z_ref, zmax_ref, zmin_ref,
             *, cin, hw):
    xi = x_ref[0][:, :hw]
    wt = wt_ref[...]
    bv = bv_ref[0:1, :]
    mx = mn = None
    for j in range(K):
        xj = rows_ref[0, j, :, :hw]
        feat = jnp.concatenate([xi, xj - xi], axis=1)
        z = jnp.dot(feat, wt, preferred_element_type=jnp.float32) + bv
        z_ref[0, j] = z
        if mx is None:
            mx = z
            mn = z
        else:
            mx = jnp.maximum(mx, z)
            mn = jnp.minimum(mn, z)
    zmax_ref[0] = mx
    zmin_ref[0] = mn


def _c1(rows, x, wt, bv, cin, hw):
    return pl.pallas_call(
        functools.partial(_c1_body, cin=cin, hw=hw),
        grid=(B, N // TP),
        in_specs=[
            pl.BlockSpec((1, K, TP, 2 * CH), lambda b, t: (b, 0, t, 0)),
            pl.BlockSpec((1, TP, cin), lambda b, t: (b, t, 0)),
            pl.BlockSpec((2 * hw, CH), lambda b, t: (0, 0)),
            pl.BlockSpec((8, CH), lambda b, t: (0, 0)),
        ],
        out_specs=(
            pl.BlockSpec((1, K, TP, CH), lambda b, t: (b, 0, t, 0)),
            pl.BlockSpec((1, TP, CH), lambda b, t: (b, t, 0)),
            pl.BlockSpec((1, TP, CH), lambda b, t: (b, t, 0)),
        ),
        out_shape=(
            jax.ShapeDtypeStruct((B, K, N, CH), jnp.float32),
            jax.ShapeDtypeStruct((B, N, CH), jnp.float32),
            jax.ShapeDtypeStruct((B, N, CH), jnp.float32),
        ),
    )(rows, x, wt, bv)


def _c2_body(zmax_ref, zmin_ref, mean_ref, var_ref, g_ref, be_ref, res_ref,
             out_ref):
    mean = mean_ref[0, :]
    var = var_ref[0, :]
    s = jnp.sqrt(var + 1e-5)
    g = g_ref[0, :]
    be = be_ref[0, :]
    sel = jnp.where((g >= 0.0)[None, :], zmax_ref[0], zmin_ref[0])
    out = jnp.maximum((sel - mean[None, :]) / s[None, :] * g[None, :]
                      + be[None, :], 0.0)
    out_ref[0] = out + res_ref[0]


def _c2(zmax, zmin, mean2, var2, g2, be2, res):
    return pl.pallas_call(
        _c2_body,
        grid=(B, N // TP),
        in_specs=[
            pl.BlockSpec((1, TP, CH), lambda b, t: (b, t, 0)),
            pl.BlockSpec((1, TP, CH), lambda b, t: (b, t, 0)),
            pl.BlockSpec((8, CH), lambda b, t: (0, 0)),
            pl.BlockSpec((8, CH), lambda b, t: (0, 0)),
            pl.BlockSpec((8, CH), lambda b, t: (0, 0)),
            pl.BlockSpec((8, CH), lambda b, t: (0, 0)),
            pl.BlockSpec((1, TP, CH), lambda b, t: (b, t, 0)),
        ],
        out_specs=pl.BlockSpec((1, TP, CH), lambda b, t: (b, t, 0)),
        out_shape=jax.ShapeDtypeStruct((B, N, CH), jnp.float32),
    )(zmax, zmin, mean2, var2, g2, be2, res)


FO = 1024


FCS = 128


def _f1_body(f_ref, w_ref, bf_ref, mx_ref, mn_ref, s1_ref, s2_ref):
    w = w_ref[...]
    bf = bf_ref[0:1, :]
    mx = mn = s1 = s2 = None
    for ch in range(N // FCS):
        y = jnp.dot(f_ref[0, pl.ds(ch * FCS, FCS), :], w,
                    preferred_element_type=jnp.float32) + bf
        cmx = jnp.max(y, axis=0)
        cmn = jnp.min(y, axis=0)
        cs1 = jnp.sum(y, axis=0)
        cs2 = jnp.sum(y * y, axis=0)
        if mx is None:
            mx, mn, s1, s2 = cmx, cmn, cs1, cs2
        else:
            mx = jnp.maximum(mx, cmx)
            mn = jnp.minimum(mn, cmn)
            s1 = s1 + cs1
            s2 = s2 + cs2
    mx_ref[0] = jnp.broadcast_to(mx[None, :], (8, FO))
    mn_ref[0] = jnp.broadcast_to(mn[None, :], (8, FO))
    s1_ref[0] = jnp.broadcast_to(s1[None, :], (8, FO))
    s2_ref[0] = jnp.broadcast_to(s2[None, :], (8, FO))


def _f1(feats, wfT, bf2):
    return pl.pallas_call(
        _f1_body,
        grid=(B,),
        in_specs=[
            pl.BlockSpec((1, N, NB * CH), lambda b: (b, 0, 0)),
            pl.BlockSpec((NB * CH, FO), lambda b: (0, 0)),
            pl.BlockSpec((8, FO), lambda b: (0, 0)),
        ],
        out_specs=tuple(pl.BlockSpec((1, 8, FO), lambda b: (b, 0, 0))
                        for _ in range(4)),
        out_shape=tuple(jax.ShapeDtypeStruct((B, 8, FO), jnp.float32)
                        for _ in range(4)),
    )(feats, wfT, bf2)


def _f2_body(mx_ref, mn_ref, s1_ref, s2_ref, g_ref, be_ref, out_ref):
    ssum = jnp.sum(s1_ref[:, 0, :], axis=0)
    ssq = jnp.sum(s2_ref[:, 0, :], axis=0)
    mean = ssum / CNT_F
    var = ssq / CNT_F - mean * mean
    s = jnp.sqrt(var + 1e-5)
    g = g_ref[0, :]
    be = be_ref[0, :]
    sel = jnp.where((g >= 0.0)[None, None, :], mx_ref[...], mn_ref[...])
    out_ref[...] = jnp.maximum(
        (sel - mean[None, None, :]) / s[None, None, :] * g[None, None, :]
        + be[None, None, :], 0.0)


def _f2(mx, mn, s1, s2, g2, be2):
    return pl.pallas_call(
        _f2_body,
        out_shape=jax.ShapeDtypeStruct((B, 8, FO), jnp.float32),
    )(mx, mn, s1, s2, g2, be2)


def _edge_block(x, p, d, res, cin):
    W, bb, g, be = p
    half = W.shape[1] // 2
    wt = W.T
    idxg = _knn(x, d)
    table = jnp.pad(x.reshape(B * N, cin), ((0, 0), (0, 2 * CH - cin)))
    rows = _gather(idxg.reshape(_L_TOT), table)
    bv = jnp.broadcast_to(bb[None, :], (8, CH))
    zfull, zmax, zmin = _c1(rows.reshape(B, K, N, 2 * CH), x, wt, bv,
                            cin, half)
    z4 = jax.lax.optimization_barrier(
        jnp.transpose(zfull, (0, 3, 2, 1)))
    mean = jnp.mean(z4, axis=(0, 2, 3))
    var = jnp.var(z4, axis=(0, 2, 3))
    mean2 = jnp.broadcast_to(mean[None, :], (8, CH))
    var2 = jnp.broadcast_to(var[None, :], (8, CH))
    g2 = jnp.broadcast_to(g[None, :], (8, CH))
    be2 = jnp.broadcast_to(be[None, :], (8, CH))
    return _c2(zmax, zmin, mean2, var2, g2, be2, res)


def kernel(inputs, params):
    x0 = jnp.transpose(inputs[:, :, :, 0], (0, 2, 1))
    x0 = jnp.pad(x0, ((0, 0), (0, 0), (0, 5)))
    zeros = jnp.zeros((B, N, CH), jnp.float32)
    x = _edge_block(x0, params['head'], 1, zeros, 8)
    feats = [x]
    for i in range(NB - 1):
        d = i % 4 + 1
        x = _edge_block(x, params['blocks'][i], d, x, CH)
        feats.append(x)
    feats_pm = jnp.concatenate(feats, axis=2)

    Wf, bf, gf, bef = params['fusion']
    wfT = Wf.T
    bf2 = jnp.broadcast_to(bf[None, :], (8, FO))
    mx, mn, s1, s2 = _f1(feats_pm, wfT, bf2)
    gf2 = jnp.broadcast_to(gf[None, :], (8, FO))
    bef2 = jnp.broadcast_to(bef[None, :], (8, FO))
    fus = _f2(mx, mn, s1, s2, gf2, bef2)[:, 0, :]

    fus_big = jnp.broadcast_to(fus[:, :, None, None], (B, FO, N, 1))
    feats_cm = jnp.transpose(feats_pm, (0, 2, 1))[:, :, :, None]
    return jnp.concatenate([fus_big, feats_cm], axis=1)
